# baseline (device time: 12449 ns/iter reference)
import jax
import jax.numpy as jnp
from jax import lax
from jax.experimental import pallas as pl
from jax.experimental.pallas import tpu as pltpu

N_PIECES = 16


def kernel(ids, E):
    v_local, d = E.shape
    t = ids.shape[0]
    t_half = t // 2
    rpp = t_half // N_PIECES
    shift = v_local.bit_length() - 1

    my_x = lax.axis_index("x")

    ids_chunk = lax.dynamic_slice(ids.astype(jnp.int32), (my_x * t_half,), (t_half,))
    rows = E[ids_chunk & (v_local - 1)]
    ids2d = ids_chunk.reshape(t_half, 1)

    def body(p_hbm_ref, i_ref, out_ref, pf_ref, pbf_ref, yrecv_ref, pm_ref,
             q_ref, xsend_ref, xrecv_ref, load_sem, y_send, y_recv,
             x_send, x_recv):
        mx = lax.axis_index("x")
        my = lax.axis_index("y")
        mz = lax.axis_index("z")
        y_partner = (mx, 1 - my, mz)
        x_partner = (1 - mx, my, mz)

        load = pltpu.make_async_copy(p_hbm_ref, pf_ref, load_sem)
        load.start()

        barrier = pltpu.get_barrier_semaphore()
        for nbr in (y_partner, x_partner):
            pl.semaphore_signal(
                barrier, inc=1, device_id=nbr,
                device_id_type=pl.DeviceIdType.MESH,
            )
        pl.semaphore_wait(barrier, 2)
        load.wait()

        my_rows = mx * t_half
        other_rows = (1 - mx) * t_half

        y_rdmas = []
        for k in range(N_PIECES):
            sl = pl.ds(k * rpp, rpp)
            pbf_ref[sl, :] = pf_ref[sl, :].astype(jnp.bfloat16)
            r = pltpu.make_async_remote_copy(
                src_ref=pbf_ref.at[sl],
                dst_ref=yrecv_ref.at[sl],
                send_sem=y_send.at[k],
                recv_sem=y_recv.at[k],
                device_id=y_partner,
                device_id_type=pl.DeviceIdType.MESH,
            )
            r.start()
            y_rdmas.append(r)

        m = ((i_ref[:, :] >> shift) == my).astype(jnp.float32)
        pm_ref[:, :] = pf_ref[:, :] * m
        q_ref[:, :] = 1.0 - m

        x_rdmas = []
        for k in range(N_PIECES):
            sl = pl.ds(k * rpp, rpp)
            out_sl = pl.ds(my_rows + k * rpp, rpp)
            y_rdmas[k].wait_recv()
            s = pm_ref[sl, :] + yrecv_ref[sl, :].astype(jnp.float32) * q_ref[sl, :]
            out_ref[out_sl, :] = s
            xsend_ref[sl, :] = s.astype(jnp.bfloat16)
            r = pltpu.make_async_remote_copy(
                src_ref=xsend_ref.at[sl],
                dst_ref=xrecv_ref.at[sl],
                send_sem=x_send.at[k],
                recv_sem=x_recv.at[k],
                device_id=x_partner,
                device_id_type=pl.DeviceIdType.MESH,
            )
            r.start()
            x_rdmas.append(r)

        for k in range(N_PIECES):
            sl = pl.ds(k * rpp, rpp)
            out_sl = pl.ds(other_rows + k * rpp, rpp)
            y_rdmas[k].wait_send()
            x_rdmas[k].wait()
            out_ref[out_sl, :] = xrecv_ref[sl, :].astype(jnp.float32)

    return pl.pallas_call(
        body,
        out_shape=jax.ShapeDtypeStruct((t, d), jnp.float32),
        in_specs=[
            pl.BlockSpec(memory_space=pltpu.MemorySpace.HBM),
            pl.BlockSpec(memory_space=pltpu.VMEM),
        ],
        out_specs=pl.BlockSpec(memory_space=pltpu.VMEM),
        scratch_shapes=[
            pltpu.VMEM((t_half, d), jnp.float32),
            pltpu.VMEM((t_half, d), jnp.bfloat16),
            pltpu.VMEM((t_half, d), jnp.bfloat16),
            pltpu.VMEM((t_half, d), jnp.float32),
            pltpu.VMEM((t_half, 1), jnp.float32),
            pltpu.VMEM((t_half, d), jnp.bfloat16),
            pltpu.VMEM((t_half, d), jnp.bfloat16),
            pltpu.SemaphoreType.DMA,
            pltpu.SemaphoreType.DMA((N_PIECES,)),
            pltpu.SemaphoreType.DMA((N_PIECES,)),
            pltpu.SemaphoreType.DMA((N_PIECES,)),
            pltpu.SemaphoreType.DMA((N_PIECES,)),
        ],
        compiler_params=pltpu.CompilerParams(collective_id=0),
    )(rows, ids2d)


# device time: 12132 ns/iter; 1.0261x vs baseline; 1.0261x over previous
import jax
import jax.numpy as jnp
from jax import lax
from jax.experimental import pallas as pl
from jax.experimental.pallas import tpu as pltpu

N_PIECES = 8


def kernel(ids, E):
    v_local, d = E.shape
    t = ids.shape[0]
    t_half = t // 2
    rpp = t_half // N_PIECES
    shift = v_local.bit_length() - 1

    my_x = lax.axis_index("x")

    ids_chunk = lax.dynamic_slice(ids.astype(jnp.int32), (my_x * t_half,), (t_half,))
    rows = E[ids_chunk & (v_local - 1)]
    ids2d = ids_chunk.reshape(t_half, 1)

    def body(p_hbm_ref, i_ref, out_ref, pf_ref, pbf_ref, yrecv_ref, pm_ref,
             q_ref, xsend_ref, xrecv_ref, load_sem, y_send, y_recv,
             x_send, x_recv):
        mx = lax.axis_index("x")
        my = lax.axis_index("y")
        mz = lax.axis_index("z")
        y_partner = (mx, 1 - my, mz)
        x_partner = (1 - mx, my, mz)

        load = pltpu.make_async_copy(p_hbm_ref, pf_ref, load_sem)
        load.start()

        barrier = pltpu.get_barrier_semaphore()
        for nbr in (y_partner, x_partner):
            pl.semaphore_signal(
                barrier, inc=1, device_id=nbr,
                device_id_type=pl.DeviceIdType.MESH,
            )
        pl.semaphore_wait(barrier, 2)
        load.wait()

        my_rows = mx * t_half
        other_rows = (1 - mx) * t_half

        y_rdmas = []
        for k in range(N_PIECES):
            sl = pl.ds(k * rpp, rpp)
            pbf_ref[sl, :] = pf_ref[sl, :].astype(jnp.bfloat16)
            r = pltpu.make_async_remote_copy(
                src_ref=pbf_ref.at[sl],
                dst_ref=yrecv_ref.at[sl],
                send_sem=y_send.at[k],
                recv_sem=y_recv.at[k],
                device_id=y_partner,
                device_id_type=pl.DeviceIdType.MESH,
            )
            r.start()
            y_rdmas.append(r)

        m = ((i_ref[:, :] >> shift) == my).astype(jnp.float32)
        pm_ref[:, :] = pf_ref[:, :] * m
        q_ref[:, :] = 1.0 - m

        x_rdmas = []
        for k in range(N_PIECES):
            sl = pl.ds(k * rpp, rpp)
            out_sl = pl.ds(my_rows + k * rpp, rpp)
            y_rdmas[k].wait_recv()
            s = pm_ref[sl, :] + yrecv_ref[sl, :].astype(jnp.float32) * q_ref[sl, :]
            out_ref[out_sl, :] = s
            xsend_ref[sl, :] = s.astype(jnp.bfloat16)
            r = pltpu.make_async_remote_copy(
                src_ref=xsend_ref.at[sl],
                dst_ref=xrecv_ref.at[sl],
                send_sem=x_send.at[k],
                recv_sem=x_recv.at[k],
                device_id=x_partner,
                device_id_type=pl.DeviceIdType.MESH,
            )
            r.start()
            x_rdmas.append(r)

        for k in range(N_PIECES):
            sl = pl.ds(k * rpp, rpp)
            out_sl = pl.ds(other_rows + k * rpp, rpp)
            y_rdmas[k].wait_send()
            x_rdmas[k].wait()
            out_ref[out_sl, :] = xrecv_ref[sl, :].astype(jnp.float32)

    return pl.pallas_call(
        body,
        out_shape=jax.ShapeDtypeStruct((t, d), jnp.float32),
        in_specs=[
            pl.BlockSpec(memory_space=pltpu.MemorySpace.HBM),
            pl.BlockSpec(memory_space=pltpu.VMEM),
        ],
        out_specs=pl.BlockSpec(memory_space=pltpu.VMEM),
        scratch_shapes=[
            pltpu.VMEM((t_half, d), jnp.float32),
            pltpu.VMEM((t_half, d), jnp.bfloat16),
            pltpu.VMEM((t_half, d), jnp.bfloat16),
            pltpu.VMEM((t_half, d), jnp.float32),
            pltpu.VMEM((t_half, 1), jnp.float32),
            pltpu.VMEM((t_half, d), jnp.bfloat16),
            pltpu.VMEM((t_half, d), jnp.bfloat16),
            pltpu.SemaphoreType.DMA,
            pltpu.SemaphoreType.DMA((N_PIECES,)),
            pltpu.SemaphoreType.DMA((N_PIECES,)),
            pltpu.SemaphoreType.DMA((N_PIECES,)),
            pltpu.SemaphoreType.DMA((N_PIECES,)),
        ],
        compiler_params=pltpu.CompilerParams(collective_id=0),
    )(rows, ids2d)
